# baseline (device time: 51970 ns/iter reference)
import jax
import jax.numpy as jnp
from jax import lax
from jax.experimental import pallas as pl
from jax.experimental.pallas import tpu as pltpu

N_DEV = 4


def kernel(x, Win0, Wout0, Win1, Wout1, Win2, Wout2):
    b, d = x.shape
    h_sh = Win0.shape[1]

    def body(x_ref, wi0_ref, wo0_ref, wi1_ref, wo1_ref, wi2_ref, wo2_ref,
             out_ref,
             x_stage, x_recv,
             wi_mine, wo_mine, wi_cross, wo_cross,
             p_send, p_recv, y_stage, y_recv,
             w_send, w_recv, x_send_sem, x_recv_sem,
             p_send_sems, p_recv_sems, y_send_sems, y_recv_sems):
        my = lax.axis_index("i")
        myslot = lax.rem(my, 2)
        pair = my + 1 - 2 * myslot
        cross = N_DEV - 1 - my
        crosspair = 2 - my + 2 * myslot

        barrier = pltpu.get_barrier_semaphore()
        for nbr in (pair, cross):
            pl.semaphore_signal(barrier, inc=1, device_id=(nbr,),
                                device_id_type=pl.DeviceIdType.MESH)
        pl.semaphore_wait(barrier, 2)

        wi_refs = [wi0_ref, wi1_ref, wi2_ref]
        wo_refs = [wo0_ref, wo1_ref, wo2_ref]
        w_rdmas = []
        for l in range(3):
            wi_mine[l, :, :] = wi_refs[l][...].astype(jnp.bfloat16)
            wo_mine[l, :, :] = wo_refs[l][...].astype(jnp.bfloat16)
            r_wi = pltpu.make_async_remote_copy(
                src_ref=wi_mine.at[l], dst_ref=wi_cross.at[l],
                send_sem=w_send.at[l], recv_sem=w_recv.at[l],
                device_id=(cross,), device_id_type=pl.DeviceIdType.MESH,
            )
            r_wi.start()
            r_wo = pltpu.make_async_remote_copy(
                src_ref=wo_mine.at[l], dst_ref=wo_cross.at[l],
                send_sem=w_send.at[3 + l], recv_sem=w_recv.at[3 + l],
                device_id=(cross,), device_id_type=pl.DeviceIdType.MESH,
            )
            r_wo.start()
            w_rdmas.append((r_wi, r_wo))

        x_stage[...] = x_ref[...].astype(jnp.bfloat16)
        x_rdma = pltpu.make_async_remote_copy(
            src_ref=x_stage, dst_ref=x_recv,
            send_sem=x_send_sem.at[0], recv_sem=x_recv_sem.at[0],
            device_id=(pair,), device_id_type=pl.DeviceIdType.MESH,
        )
        x_rdma.start()

        def finish_partial(l, c, h1, h2):
            p = (
                jnp.dot(h1, wo_mine[l], preferred_element_type=jnp.float32)
                + jnp.dot(h2, wo_cross[l], preferred_element_type=jnp.float32)
            )
            p_send[l, c, :, :] = p.astype(jnp.bfloat16)
            r = pltpu.make_async_remote_copy(
                src_ref=p_send.at[l, c], dst_ref=p_recv.at[l, 1 - c],
                send_sem=p_send_sems.at[l, c],
                recv_sem=p_recv_sems.at[l, 1 - c],
                device_id=(pair,), device_id_type=pl.DeviceIdType.MESH,
            )
            r.start()
            return r

        def compute_send(l, c, a):
            h1 = jnp.dot(a, wi_mine[l], preferred_element_type=jnp.float32)
            h2 = jnp.dot(a, wi_cross[l], preferred_element_type=jnp.float32)
            h1 = jnp.maximum(h1, 0.0).astype(jnp.bfloat16)
            h2 = jnp.maximum(h2, 0.0).astype(jnp.bfloat16)
            return finish_partial(l, c, h1, h2)

        def reduce(l, c, rds):
            rds[(l, 1 - c)].wait_recv()
            rds[(l, c)].wait_send()
            return (p_send[l, c].astype(jnp.float32)
                    + p_recv[l, c].astype(jnp.float32))

        rds = {}
        a00 = x_stage[...]
        h1_00 = jnp.dot(a00, wi_mine[0], preferred_element_type=jnp.float32)
        h1_00 = jnp.maximum(h1_00, 0.0).astype(jnp.bfloat16)
        r_wi0, r_wo0 = w_rdmas[0]
        r_wi0.wait()
        h2_00 = jnp.dot(a00, wi_cross[0], preferred_element_type=jnp.float32)
        h2_00 = jnp.maximum(h2_00, 0.0).astype(jnp.bfloat16)
        r_wo0.wait()
        rds[(0, 0)] = finish_partial(0, 0, h1_00, h2_00)

        x_rdma.wait()
        rds[(0, 1)] = compute_send(0, 1, x_recv[...])

        for l in (1, 2):
            w_rdmas[l][0].wait()
            w_rdmas[l][1].wait()
            for c in (0, 1):
                xn = reduce(l - 1, c, rds)
                rds[(l, c)] = compute_send(l, c, xn.astype(jnp.bfloat16))

        y_rdmas = []
        row_of = [my * b, pair * b]
        for c in (0, 1):
            xn = reduce(2, c, rds)
            out_ref[pl.ds(row_of[c], b), :] = xn
            y_stage[c, :, :] = xn.astype(jnp.bfloat16)
            yr = pltpu.make_async_remote_copy(
                src_ref=y_stage.at[c], dst_ref=y_recv.at[c],
                send_sem=y_send_sems.at[c], recv_sem=y_recv_sems.at[c],
                device_id=(cross,), device_id_type=pl.DeviceIdType.MESH,
            )
            yr.start()
            y_rdmas.append(yr)
        other_row_of = [cross * b, crosspair * b]
        for c in (0, 1):
            y_rdmas[c].wait()
            out_ref[pl.ds(other_row_of[c], b), :] = (
                y_recv[c].astype(jnp.float32))

    return pl.pallas_call(
        body,
        out_shape=jax.ShapeDtypeStruct((N_DEV * b, d), jnp.float32),
        in_specs=[pl.BlockSpec(memory_space=pltpu.VMEM)] * 7,
        out_specs=pl.BlockSpec(memory_space=pltpu.VMEM),
        scratch_shapes=[
            pltpu.VMEM((b, d), jnp.bfloat16),
            pltpu.VMEM((b, d), jnp.bfloat16),
            pltpu.VMEM((3, d, h_sh), jnp.bfloat16),
            pltpu.VMEM((3, h_sh, d), jnp.bfloat16),
            pltpu.VMEM((3, d, h_sh), jnp.bfloat16),
            pltpu.VMEM((3, h_sh, d), jnp.bfloat16),
            pltpu.VMEM((3, 2, b, d), jnp.bfloat16),
            pltpu.VMEM((3, 2, b, d), jnp.bfloat16),
            pltpu.VMEM((2, b, d), jnp.bfloat16),
            pltpu.VMEM((2, b, d), jnp.bfloat16),
            pltpu.SemaphoreType.DMA((6,)),
            pltpu.SemaphoreType.DMA((6,)),
            pltpu.SemaphoreType.DMA((1,)),
            pltpu.SemaphoreType.DMA((1,)),
            pltpu.SemaphoreType.DMA((3, 2)),
            pltpu.SemaphoreType.DMA((3, 2)),
            pltpu.SemaphoreType.DMA((2,)),
            pltpu.SemaphoreType.DMA((2,)),
        ],
        compiler_params=pltpu.CompilerParams(collective_id=0),
    )(x, Win0, Wout0, Win1, Wout1, Win2, Wout2)
